# initial kernel scaffold (unmeasured)
import jax
import jax.numpy as jnp
from jax import lax
from jax.experimental import pallas as pl
from jax.experimental.pallas import tpu as pltpu


def kernel(
    x,
):
    def body(*refs):
        pass

    out_shape = jax.ShapeDtypeStruct(..., jnp.float32)
    return pl.pallas_call(body, out_shape=out_shape)(...)


# baseline (device time: 54931 ns/iter reference)
import jax
import jax.numpy as jnp
from jax import lax
from jax.experimental import pallas as pl
from jax.experimental.pallas import tpu as pltpu


def kernel(x):
    m, n = x.shape

    def body(x_ref, out_ref, peer_x_ref, sum_ref, peer_y_ref,
             sem_sx, sem_rx, sem_sy, sem_ry):
        my_x = lax.axis_index("x")
        my_y = lax.axis_index("y")
        other_x = 1 - my_x
        other_y = 1 - my_y

        barrier_sem = pltpu.get_barrier_semaphore()
        pl.semaphore_signal(barrier_sem, inc=1, device_id=(other_x, my_y),
                            device_id_type=pl.DeviceIdType.MESH)
        pl.semaphore_signal(barrier_sem, inc=1, device_id=(my_x, other_y),
                            device_id_type=pl.DeviceIdType.MESH)
        pl.semaphore_wait(barrier_sem, 2)

        rdma_x = pltpu.make_async_remote_copy(
            src_ref=x_ref,
            dst_ref=peer_x_ref,
            send_sem=sem_sx,
            recv_sem=sem_rx,
            device_id=(other_x, my_y),
            device_id_type=pl.DeviceIdType.MESH,
        )
        rdma_x.start()
        rdma_x.wait()

        sum_ref[...] = x_ref[...] + peer_x_ref[...]
        out_ref[:, pl.ds(my_y * n, n)] = sum_ref[...]

        rdma_y = pltpu.make_async_remote_copy(
            src_ref=sum_ref,
            dst_ref=peer_y_ref,
            send_sem=sem_sy,
            recv_sem=sem_ry,
            device_id=(my_x, other_y),
            device_id_type=pl.DeviceIdType.MESH,
        )
        rdma_y.start()
        rdma_y.wait()

        out_ref[:, pl.ds(other_y * n, n)] = peer_y_ref[...]

    return pl.pallas_call(
        body,
        out_shape=jax.ShapeDtypeStruct((m, 2 * n), x.dtype),
        in_specs=[pl.BlockSpec(memory_space=pltpu.VMEM)],
        out_specs=pl.BlockSpec(memory_space=pltpu.VMEM),
        scratch_shapes=[
            pltpu.VMEM((m, n), x.dtype),
            pltpu.VMEM((m, n), x.dtype),
            pltpu.VMEM((m, n), x.dtype),
            pltpu.SemaphoreType.DMA,
            pltpu.SemaphoreType.DMA,
            pltpu.SemaphoreType.DMA,
            pltpu.SemaphoreType.DMA,
        ],
        compiler_params=pltpu.CompilerParams(collective_id=0),
    )(x)


# device time: 34842 ns/iter; 1.5766x vs baseline; 1.5766x over previous
import jax
import jax.numpy as jnp
from jax import lax
from jax.experimental import pallas as pl
from jax.experimental.pallas import tpu as pltpu

C = 8


def kernel(x):
    m, n = x.shape
    mc = m // C

    def body(x_ref, out_ref, peer_x_ref, sx, rx, sy, ry):
        my_x = lax.axis_index("x")
        my_y = lax.axis_index("y")
        other_x = 1 - my_x
        other_y = 1 - my_y
        my_col = my_y * n

        barrier_sem = pltpu.get_barrier_semaphore()
        pl.semaphore_signal(barrier_sem, inc=1, device_id=(other_x, my_y),
                            device_id_type=pl.DeviceIdType.MESH)
        pl.semaphore_signal(barrier_sem, inc=1, device_id=(my_x, other_y),
                            device_id_type=pl.DeviceIdType.MESH)
        pl.semaphore_wait(barrier_sem, 2)

        x_rdmas = []
        for c in range(C):
            rows = pl.ds(c * mc, mc)
            r = pltpu.make_async_remote_copy(
                src_ref=x_ref.at[rows],
                dst_ref=peer_x_ref.at[rows],
                send_sem=sx.at[c],
                recv_sem=rx.at[c],
                device_id=(other_x, my_y),
                device_id_type=pl.DeviceIdType.MESH,
            )
            r.start()
            x_rdmas.append(r)

        y_rdmas = []
        for c in range(C):
            rows = pl.ds(c * mc, mc)
            x_rdmas[c].wait_recv()
            out_ref[rows, pl.ds(my_col, n)] = x_ref[rows, :] + peer_x_ref[rows, :]
            r = pltpu.make_async_remote_copy(
                src_ref=out_ref.at[rows, pl.ds(my_col, n)],
                dst_ref=out_ref.at[rows, pl.ds(my_col, n)],
                send_sem=sy.at[c],
                recv_sem=ry.at[c],
                device_id=(my_x, other_y),
                device_id_type=pl.DeviceIdType.MESH,
            )
            r.start()
            y_rdmas.append(r)

        for c in range(C):
            y_rdmas[c].wait_recv()
        for c in range(C):
            x_rdmas[c].wait_send()
            y_rdmas[c].wait_send()

    return pl.pallas_call(
        body,
        out_shape=jax.ShapeDtypeStruct((m, 2 * n), x.dtype),
        in_specs=[pl.BlockSpec(memory_space=pltpu.VMEM)],
        out_specs=pl.BlockSpec(memory_space=pltpu.VMEM),
        scratch_shapes=[
            pltpu.VMEM((m, n), x.dtype),
            pltpu.SemaphoreType.DMA((C,)),
            pltpu.SemaphoreType.DMA((C,)),
            pltpu.SemaphoreType.DMA((C,)),
            pltpu.SemaphoreType.DMA((C,)),
        ],
        compiler_params=pltpu.CompilerParams(collective_id=0),
    )(x)
